# baseline (speedup 1.0000x reference)
import math

import jax
import jax.numpy as jnp
import numpy as np
from jax.experimental import pallas as pl
from jax.experimental.pallas import tpu as pltpu

_NUM_TIMESTEPS = 1000


def _schedule_tables():
    steps = _NUM_TIMESTEPS + 1
    x = np.linspace(0, _NUM_TIMESTEPS, steps, dtype=np.float64)
    s = 0.008
    alphas_cumprod = np.cos((x / _NUM_TIMESTEPS + s) / (1 + s) * math.pi * 0.5) ** 2
    alphas_cumprod = alphas_cumprod / alphas_cumprod[0]
    betas = np.clip(1 - alphas_cumprod[1:] / alphas_cumprod[:-1], 0, 0.999)
    ac = np.cumprod(1.0 - betas, axis=0)
    sqrt_ac = np.sqrt(ac).astype(np.float32)
    sqrt_om = np.sqrt(1.0 - ac).astype(np.float32)
    return sqrt_ac, sqrt_om


_SQRT_AC, _SQRT_OM = _schedule_tables()


def _body(ts_ref, ta_ref, tb_ref, x_ref, n_ref, o_ref):
    b = pl.program_id(0)
    t = ts_ref[b]
    a = ta_ref[t]
    s = tb_ref[t]
    o_ref[...] = a * x_ref[...] + s * n_ref[...]


def kernel(x_start, noise, timesteps):
    B, C, H, W = x_start.shape
    R = C * H
    x3 = x_start.reshape(B, R, W)
    n3 = noise.reshape(B, R, W)
    ts = timesteps.astype(jnp.int32)
    ta = jnp.asarray(_SQRT_AC)
    tb = jnp.asarray(_SQRT_OM)

    grid_spec = pltpu.PrefetchScalarGridSpec(
        num_scalar_prefetch=3,
        grid=(B,),
        in_specs=[
            pl.BlockSpec((1, R, W), lambda b, *_: (b, 0, 0)),
            pl.BlockSpec((1, R, W), lambda b, *_: (b, 0, 0)),
        ],
        out_specs=pl.BlockSpec((1, R, W), lambda b, *_: (b, 0, 0)),
    )
    out = pl.pallas_call(
        _body,
        grid_spec=grid_spec,
        out_shape=jax.ShapeDtypeStruct((B, R, W), x_start.dtype),
    )(ts, ta, tb, x3, n3)
    return out.reshape(B, C, H, W)


# speedup vs baseline: 1.3728x; 1.3728x over previous
import math

import jax
import jax.numpy as jnp
import numpy as np
from jax.experimental import pallas as pl
from jax.experimental.pallas import tpu as pltpu

_NUM_TIMESTEPS = 1000


def _schedule_tables():
    steps = _NUM_TIMESTEPS + 1
    x = np.linspace(0, _NUM_TIMESTEPS, steps, dtype=np.float64)
    s = 0.008
    alphas_cumprod = np.cos((x / _NUM_TIMESTEPS + s) / (1 + s) * math.pi * 0.5) ** 2
    alphas_cumprod = alphas_cumprod / alphas_cumprod[0]
    betas = np.clip(1 - alphas_cumprod[1:] / alphas_cumprod[:-1], 0, 0.999)
    ac = np.cumprod(1.0 - betas, axis=0)
    sqrt_ac = np.sqrt(ac).astype(np.float32)
    sqrt_om = np.sqrt(1.0 - ac).astype(np.float32)
    return sqrt_ac, sqrt_om


_SQRT_AC, _SQRT_OM = _schedule_tables()


_NB = 4


def _body(ts_ref, ta_ref, tb_ref, x_ref, n_ref, o_ref):
    g = pl.program_id(0)
    a = jnp.stack([ta_ref[ts_ref[g * _NB + j]] for j in range(_NB)])
    s = jnp.stack([tb_ref[ts_ref[g * _NB + j]] for j in range(_NB)])
    a = a.reshape(_NB, 1, 1)
    s = s.reshape(_NB, 1, 1)
    o_ref[...] = a * x_ref[...] + s * n_ref[...]


def kernel(x_start, noise, timesteps):
    B, C, H, W = x_start.shape
    R = C * H
    x3 = x_start.reshape(B, R, W)
    n3 = noise.reshape(B, R, W)
    ts = timesteps.astype(jnp.int32)
    ta = jnp.asarray(_SQRT_AC)
    tb = jnp.asarray(_SQRT_OM)

    grid_spec = pltpu.PrefetchScalarGridSpec(
        num_scalar_prefetch=3,
        grid=(B // _NB,),
        in_specs=[
            pl.BlockSpec((_NB, R, W), lambda b, *_: (b, 0, 0)),
            pl.BlockSpec((_NB, R, W), lambda b, *_: (b, 0, 0)),
        ],
        out_specs=pl.BlockSpec((_NB, R, W), lambda b, *_: (b, 0, 0)),
    )
    out = pl.pallas_call(
        _body,
        grid_spec=grid_spec,
        out_shape=jax.ShapeDtypeStruct((B, R, W), x_start.dtype),
    )(ts, ta, tb, x3, n3)
    return out.reshape(B, C, H, W)


# speedup vs baseline: 1.3888x; 1.0116x over previous
import math

import jax
import jax.numpy as jnp
import numpy as np
from jax.experimental import pallas as pl
from jax.experimental.pallas import tpu as pltpu

_NUM_TIMESTEPS = 1000


def _schedule_tables():
    steps = _NUM_TIMESTEPS + 1
    x = np.linspace(0, _NUM_TIMESTEPS, steps, dtype=np.float64)
    s = 0.008
    alphas_cumprod = np.cos((x / _NUM_TIMESTEPS + s) / (1 + s) * math.pi * 0.5) ** 2
    alphas_cumprod = alphas_cumprod / alphas_cumprod[0]
    betas = np.clip(1 - alphas_cumprod[1:] / alphas_cumprod[:-1], 0, 0.999)
    ac = np.cumprod(1.0 - betas, axis=0)
    sqrt_ac = np.sqrt(ac).astype(np.float32)
    sqrt_om = np.sqrt(1.0 - ac).astype(np.float32)
    return sqrt_ac, sqrt_om


_SQRT_AC, _SQRT_OM = _schedule_tables()


_NB = 8


def _body(ts_ref, ta_ref, tb_ref, x_ref, n_ref, o_ref):
    g = pl.program_id(0)
    a = jnp.stack([ta_ref[ts_ref[g * _NB + j]] for j in range(_NB)])
    s = jnp.stack([tb_ref[ts_ref[g * _NB + j]] for j in range(_NB)])
    a = a.reshape(_NB, 1, 1)
    s = s.reshape(_NB, 1, 1)
    o_ref[...] = a * x_ref[...] + s * n_ref[...]


def kernel(x_start, noise, timesteps):
    B, C, H, W = x_start.shape
    R = C * H
    x3 = x_start.reshape(B, R, W)
    n3 = noise.reshape(B, R, W)
    ts = timesteps.astype(jnp.int32)
    ta = jnp.asarray(_SQRT_AC)
    tb = jnp.asarray(_SQRT_OM)

    grid_spec = pltpu.PrefetchScalarGridSpec(
        num_scalar_prefetch=3,
        grid=(B // _NB,),
        in_specs=[
            pl.BlockSpec((_NB, R, W), lambda b, *_: (b, 0, 0)),
            pl.BlockSpec((_NB, R, W), lambda b, *_: (b, 0, 0)),
        ],
        out_specs=pl.BlockSpec((_NB, R, W), lambda b, *_: (b, 0, 0)),
    )
    out = pl.pallas_call(
        _body,
        grid_spec=grid_spec,
        out_shape=jax.ShapeDtypeStruct((B, R, W), x_start.dtype),
    )(ts, ta, tb, x3, n3)
    return out.reshape(B, C, H, W)
